# baseline (speedup 1.0000x reference)
import functools

import jax
import jax.numpy as jnp
from jax import lax
from jax.experimental import pallas as pl
from jax.experimental.pallas import tpu as pltpu
from jax.experimental.pallas import tpu_sc as plsc

E = 8
K = 2
TM = 256
LANES = 16
NC, NS = 2, 16
NW = NC * NS

@functools.lru_cache(maxsize=None)
def _mesh():
    return plsc.VectorSubcoreMesh(core_axis_name="c", subcore_axis_name="s",
                                  num_cores=NC, num_subcores=NS)


def _wid():
    return lax.axis_index("s") * NC + lax.axis_index("c")


def _router_body(x_ref, gw_ref, logits_ref, ids_ref, wts_ref):
    x = x_ref[...]
    logits = jnp.dot(x, gw_ref[...], preferred_element_type=jnp.float32)
    logits_ref[...] = logits
    neg = jnp.float32(-1e30)
    lane = lax.broadcasted_iota(jnp.int32, logits.shape, 1)
    ml = jnp.where(lane < E, logits, neg)
    m1 = jnp.max(ml, axis=1, keepdims=True)
    e1 = jnp.min(jnp.where(ml == m1, lane, 127), axis=1, keepdims=True)
    ml2 = jnp.where(lane == e1, neg, ml)
    m2 = jnp.max(ml2, axis=1, keepdims=True)
    e2 = jnp.min(jnp.where(ml2 == m2, lane, 127), axis=1, keepdims=True)
    d = jnp.exp(m2 - m1)
    w1v = 1.0 / (1.0 + d)
    w2v = d / (1.0 + d)
    ids_ref[...] = jnp.where(lane == 0, e1, jnp.where(lane == 1, e2, 0))
    wts_ref[...] = jnp.where(lane == 0, w1v, jnp.where(lane == 1, w2v, 0.0))


def _router(x, gate_w):
    t, dm = x.shape
    gwp = jnp.zeros((dm, 128), jnp.float32).at[:, :E].set(gate_w)
    return pl.pallas_call(
        _router_body,
        out_shape=(
            jax.ShapeDtypeStruct((t, 128), jnp.float32),
            jax.ShapeDtypeStruct((t, 128), jnp.int32),
            jax.ShapeDtypeStruct((t, 128), jnp.float32),
        ),
    )(x, gwp)


def _dispatch_body(na, cap, nt_pad,
                   eids_hbm, wts_hbm, rt_hbm, rw_hbm, inv_hbm, te_hbm,
                   eids_v, wts_v, rt_v, rw_v, te_v):
    @pl.when(_wid() == 0)
    def _():
        pltpu.sync_copy(eids_hbm, eids_v)
        pltpu.sync_copy(wts_hbm, wts_v)
        lanes = lax.iota(jnp.int32, LANES)

        def cnt_body(i, cnt):
            ids = eids_v[pl.ds(pl.multiple_of(i * LANES, LANES), LANES)]
            for e in range(E):
                pc = jnp.sum(jnp.where(ids == e, 1, 0))
                cnt = cnt + jnp.where(lanes == e, pc, 0)
            return cnt

        cnt = lax.fori_loop(0, na // LANES, cnt_body,
                            jnp.zeros((LANES,), jnp.int32))
        padded = ((cnt + (TM - 1)) // TM) * TM
        padded = jnp.where(lanes < E, padded, 0)
        offset = plsc.cumsum(padded) - padded

        def z_body(i, c):
            b = pl.multiple_of(i * LANES, LANES)
            rt_v[pl.ds(b, LANES)] = jnp.zeros((LANES,), jnp.int32)
            rw_v[pl.ds(b, LANES)] = jnp.zeros((LANES,), jnp.float32)
            return c

        lax.fori_loop(0, cap // LANES, z_body, 0)

        def p2_body(i, rc):
            b = pl.multiple_of(i * LANES, LANES)
            ids = eids_v[pl.ds(b, LANES)]
            ws = wts_v[pl.ds(b, LANES)]
            tok = (i * LANES + lanes) // K
            dest_all = jnp.zeros((LANES,), jnp.int32)
            for e in range(E):
                m = ids == e
                r = plsc.cumsum(jnp.where(m, 1, 0))
                off_e = jnp.sum(jnp.where(lanes == e, offset + rc, 0))
                dest = jnp.where(m, off_e + r - 1, 0)
                plsc.store_scatter(rt_v, [dest], tok, mask=m)
                plsc.store_scatter(rw_v, [dest], ws, mask=m)
                dest_all = dest_all + jnp.where(m, dest, 0)
                pc = jnp.sum(jnp.where(m, 1, 0))
                rc = rc + jnp.where(lanes == e, pc, 0)
            eids_v[pl.ds(b, LANES)] = dest_all
            return rc

        lax.fori_loop(0, na // LANES, p2_body, jnp.zeros((LANES,), jnp.int32))

        for v in range(nt_pad // LANES):
            tl = (v * LANES + lanes) * TM
            acc = jnp.full((LANES,), -1, jnp.int32)
            for e in range(E):
                off_e = jnp.sum(jnp.where(lanes == e, offset, 0))
                acc = acc + jnp.where(tl >= off_e, 1, 0)
            te_v[pl.ds(v * LANES, LANES)] = jnp.clip(acc, 0, E - 1)

        pltpu.sync_copy(rt_v, rt_hbm)
        pltpu.sync_copy(rw_v, rw_hbm)
        pltpu.sync_copy(eids_v, inv_hbm)
        pltpu.sync_copy(te_v, te_hbm)


def _dispatch(eids, wts, cap, nt_pad):
    na = eids.shape[0]
    kern = functools.partial(
        pl.kernel,
        out_type=(
            jax.ShapeDtypeStruct((cap,), jnp.int32),
            jax.ShapeDtypeStruct((cap,), jnp.float32),
            jax.ShapeDtypeStruct((na,), jnp.int32),
            jax.ShapeDtypeStruct((nt_pad,), jnp.int32),
        ),
        mesh=_mesh(),
        compiler_params=pltpu.CompilerParams(needs_layout_passes=False),
        scratch_types=[
            pltpu.VMEM((na,), jnp.int32),
            pltpu.VMEM((na,), jnp.float32),
            pltpu.VMEM((cap,), jnp.int32),
            pltpu.VMEM((cap,), jnp.float32),
            pltpu.VMEM((nt_pad,), jnp.int32),
        ],
    )
    return kern(functools.partial(_dispatch_body, na, cap, nt_pad))(eids, wts)


def _gather_body(rows_per_w, chunk, nchunk,
                 x_hbm, rt_hbm, xg_hbm, idx_v, rows_a, rows_b, sem_a, sem_b):
    base = _wid() * rows_per_w
    pltpu.sync_copy(rt_hbm.at[pl.ds(base, rows_per_w)], idx_v)
    bufs = [rows_a, rows_b]
    sems = [sem_a, sem_b]
    descs = [None, None]
    for c in range(nchunk):
        p = c % 2
        if descs[p] is not None:
            descs[p].wait()
        descs[p] = pltpu.async_copy(
            x_hbm.at[idx_v.at[pl.ds(c * chunk, chunk)]], bufs[p], sems[p])
        if c >= 1:
            q = (c - 1) % 2
            descs[q].wait()
            descs[q] = None
            pltpu.sync_copy(bufs[q],
                            xg_hbm.at[pl.ds(base + (c - 1) * chunk, chunk)])
    q = (nchunk - 1) % 2
    descs[q].wait()
    pltpu.sync_copy(bufs[q],
                    xg_hbm.at[pl.ds(base + (nchunk - 1) * chunk, chunk)])


def _gather(x, row_token, cap):
    t, dm = x.shape
    rows_per_w = cap // NW
    chunk = 48
    nchunk = rows_per_w // chunk
    kern = functools.partial(
        pl.kernel,
        out_type=jax.ShapeDtypeStruct((cap, dm), jnp.float32),
        mesh=_mesh(),
        compiler_params=pltpu.CompilerParams(needs_layout_passes=False),
        scratch_types=[
            pltpu.VMEM((rows_per_w,), jnp.int32),
            pltpu.VMEM((chunk, dm), jnp.float32),
            pltpu.VMEM((chunk, dm), jnp.float32),
            pltpu.SemaphoreType.DMA,
            pltpu.SemaphoreType.DMA,
        ],
    )
    return kern(
        functools.partial(_gather_body, rows_per_w, chunk, nchunk))(
            x, row_token)


def _ffn_body(nf, te_ref, xg_ref, w1_ref, w3_ref, w2_ref, rw_ref, out_ref,
              acc_ref):
    j = pl.program_id(0)
    i = pl.program_id(1)
    x = xg_ref[...]
    a = jnp.dot(x, w1_ref[0], preferred_element_type=jnp.float32)
    b = jnp.dot(x, w3_ref[0], preferred_element_type=jnp.float32)
    h = (a / (1.0 + jnp.exp(-a))) * b
    contrib = jnp.dot(h, w2_ref[0], preferred_element_type=jnp.float32)
    sl = pl.ds(i * TM, TM)

    @pl.when(j < nf - 1)
    def _():
        @pl.when(j == 0)
        def _():
            acc_ref[sl, :] = contrib.astype(jnp.bfloat16)

        @pl.when(j > 0)
        def _():
            acc_ref[sl, :] = (acc_ref[sl, :].astype(jnp.float32)
                              + contrib).astype(jnp.bfloat16)

    @pl.when(j == nf - 1)
    def _():
        out_ref[...] = ((acc_ref[sl, :].astype(jnp.float32) + contrib)
                        * rw_ref[0][0][:, None])


def _ffn(xg, w1, w2, w3, row_weight, tile_expert, nt):
    cap, dm = xg.shape
    dff = w1.shape[2]
    fb = 896
    nf = dff // fb
    rw3 = row_weight.reshape(nt, 1, TM)
    grid_spec = pltpu.PrefetchScalarGridSpec(
        num_scalar_prefetch=1,
        grid=(nf, nt),
        in_specs=[
            pl.BlockSpec((TM, dm), lambda j, i, te: (i, 0)),
            pl.BlockSpec((1, dm, fb), lambda j, i, te: (te[i], 0, j)),
            pl.BlockSpec((1, dm, fb), lambda j, i, te: (te[i], 0, j)),
            pl.BlockSpec((1, fb, dm), lambda j, i, te: (te[i], j, 0)),
            pl.BlockSpec((1, 1, TM), lambda j, i, te: (i, 0, 0)),
        ],
        out_specs=pl.BlockSpec(
            (TM, dm), lambda j, i, te: (jnp.where(j == nf - 1, i, 0), 0)),
        scratch_shapes=[pltpu.VMEM((cap, dm), jnp.bfloat16)],
    )
    return pl.pallas_call(
        functools.partial(_ffn_body, nf),
        grid_spec=grid_spec,
        out_shape=jax.ShapeDtypeStruct((cap, dm), jnp.float32),
    )(tile_expert, xg, w1, w3, w2, rw3)


def _combine_body(t, tok_chunk, nchunk,
                  y_hbm, inv_hbm, out_hbm, inv_v, ybuf_a, ybuf_b, obuf,
                  sem_a, sem_b):
    wid = _wid()
    tok_per_w = t // NW
    dm = obuf.shape[1]
    pltpu.sync_copy(inv_hbm.at[pl.ds(wid * tok_per_w * K, tok_per_w * K)],
                    inv_v)
    bufs = [ybuf_a, ybuf_b]
    sems = [sem_a, sem_b]

    def issue(c):
        return pltpu.async_copy(
            y_hbm.at[inv_v.at[pl.ds(c * tok_chunk * K, tok_chunk * K)]],
            bufs[c % 2], sems[c % 2])

    descs = [issue(0), issue(1)]
    for c in range(nchunk):
        descs[c % 2].wait()
        ybuf = bufs[c % 2]

        def rloop(r, cr):
            def gloop(g, cg):
                s = pl.ds(pl.multiple_of(g * LANES, LANES), LANES)
                obuf[r, s] = ybuf[2 * r, s] + ybuf[2 * r + 1, s]
                return cg
            lax.fori_loop(0, dm // LANES, gloop, 0)
            return cr

        lax.fori_loop(0, tok_chunk, rloop, 0)
        if c + 2 < nchunk:
            descs[c % 2] = issue(c + 2)
        pltpu.sync_copy(
            obuf, out_hbm.at[pl.ds(wid * tok_per_w + c * tok_chunk, tok_chunk)])


def _combine(y, inv, t):
    dm = y.shape[1]
    tok_chunk = 16
    nchunk = t // NW // tok_chunk
    kern = functools.partial(
        pl.kernel,
        out_type=jax.ShapeDtypeStruct((t, dm), jnp.float32),
        mesh=_mesh(),
        compiler_params=pltpu.CompilerParams(needs_layout_passes=False),
        scratch_types=[
            pltpu.VMEM((t // NW * K,), jnp.int32),
            pltpu.VMEM((tok_chunk * K, dm), jnp.float32),
            pltpu.VMEM((tok_chunk * K, dm), jnp.float32),
            pltpu.VMEM((tok_chunk, dm), jnp.float32),
            pltpu.SemaphoreType.DMA,
            pltpu.SemaphoreType.DMA,
        ],
    )
    return kern(functools.partial(_combine_body, t, tok_chunk, nchunk))(y, inv)


def kernel(hidden_states, gate_w, w1, w2, w3):
    batch, seq, dm = hidden_states.shape
    t = batch * seq
    x = hidden_states.reshape(t, dm)
    nt = (t * K) // TM + E
    nt_pad = ((nt + LANES - 1) // LANES) * LANES
    cap = nt * TM

    logits_p, ids_p, wts_p = _router(x, gate_w)
    router_logits = logits_p[:, :E]
    eids = ids_p[:, :K].reshape(t * K)
    wv = wts_p[:, :K].reshape(t * K)

    row_token, row_weight, inv, te = _dispatch(eids, wv, cap, nt_pad)
    xg = _gather(x, row_token, cap)
    y = _ffn(xg, w1, w2, w3, row_weight, te[:nt], nt)
    out = _combine(y, inv, t)
    return (out.reshape(batch, seq, dm), router_logits)


# speedup vs baseline: 1.2275x; 1.2275x over previous
import functools

import jax
import jax.numpy as jnp
from jax import lax
from jax.experimental import pallas as pl
from jax.experimental.pallas import tpu as pltpu
from jax.experimental.pallas import tpu_sc as plsc

E = 8
K = 2
TM = 128
LANES = 16
NC, NS = 2, 16
NW = NC * NS

@functools.lru_cache(maxsize=None)
def _mesh():
    return plsc.VectorSubcoreMesh(core_axis_name="c", subcore_axis_name="s",
                                  num_cores=NC, num_subcores=NS)


def _wid():
    return lax.axis_index("s") * NC + lax.axis_index("c")


def _router_body(x_ref, gw_ref, logits_ref, ids_ref, wts_ref):
    x = x_ref[...]
    logits = jnp.dot(x, gw_ref[...], preferred_element_type=jnp.float32)
    logits_ref[...] = logits
    neg = jnp.float32(-1e30)
    lane = lax.broadcasted_iota(jnp.int32, logits.shape, 1)
    ml = jnp.where(lane < E, logits, neg)
    m1 = jnp.max(ml, axis=1, keepdims=True)
    e1 = jnp.min(jnp.where(ml == m1, lane, 127), axis=1, keepdims=True)
    ml2 = jnp.where(lane == e1, neg, ml)
    m2 = jnp.max(ml2, axis=1, keepdims=True)
    e2 = jnp.min(jnp.where(ml2 == m2, lane, 127), axis=1, keepdims=True)
    d = jnp.exp(m2 - m1)
    w1v = 1.0 / (1.0 + d)
    w2v = d / (1.0 + d)
    ids_ref[...] = jnp.where(lane == 0, e1, jnp.where(lane == 1, e2, 0))
    wts_ref[...] = jnp.where(lane == 0, w1v, jnp.where(lane == 1, w2v, 0.0))


def _router(x, gate_w):
    t, dm = x.shape
    gwp = jnp.zeros((dm, 128), jnp.float32).at[:, :E].set(gate_w)
    return pl.pallas_call(
        _router_body,
        out_shape=(
            jax.ShapeDtypeStruct((t, 128), jnp.float32),
            jax.ShapeDtypeStruct((t, 128), jnp.int32),
            jax.ShapeDtypeStruct((t, 128), jnp.float32),
        ),
    )(x, gwp)


def _dispatch_body(na, cap, nt_pad,
                   eids_hbm, wts_hbm, rt_hbm, rw_hbm, inv_hbm, te_hbm,
                   eids_v, wts_v, rt_v, rw_v, te_v):
    @pl.when(_wid() == 0)
    def _():
        pltpu.sync_copy(eids_hbm, eids_v)
        pltpu.sync_copy(wts_hbm, wts_v)
        lanes = lax.iota(jnp.int32, LANES)

        def cnt_body(i, cnt):
            ids = eids_v[pl.ds(pl.multiple_of(i * LANES, LANES), LANES)]
            for e in range(E):
                pc = jnp.sum(jnp.where(ids == e, 1, 0))
                cnt = cnt + jnp.where(lanes == e, pc, 0)
            return cnt

        cnt = lax.fori_loop(0, na // LANES, cnt_body,
                            jnp.zeros((LANES,), jnp.int32))
        padded = ((cnt + (TM - 1)) // TM) * TM
        padded = jnp.where(lanes < E, padded, 0)
        offset = plsc.cumsum(padded) - padded

        def z_body(i, c):
            b = pl.multiple_of(i * LANES, LANES)
            rt_v[pl.ds(b, LANES)] = jnp.zeros((LANES,), jnp.int32)
            rw_v[pl.ds(b, LANES)] = jnp.zeros((LANES,), jnp.float32)
            return c

        lax.fori_loop(0, cap // LANES, z_body, 0)

        def p2_body(i, rc):
            b = pl.multiple_of(i * LANES, LANES)
            ids = eids_v[pl.ds(b, LANES)]
            ws = wts_v[pl.ds(b, LANES)]
            tok = (i * LANES + lanes) // K
            dest_all = jnp.zeros((LANES,), jnp.int32)
            for e in range(E):
                m = ids == e
                r = plsc.cumsum(jnp.where(m, 1, 0))
                off_e = jnp.sum(jnp.where(lanes == e, offset + rc, 0))
                dest = jnp.where(m, off_e + r - 1, 0)
                plsc.store_scatter(rt_v, [dest], tok, mask=m)
                plsc.store_scatter(rw_v, [dest], ws, mask=m)
                dest_all = dest_all + jnp.where(m, dest, 0)
                pc = jnp.sum(jnp.where(m, 1, 0))
                rc = rc + jnp.where(lanes == e, pc, 0)
            eids_v[pl.ds(b, LANES)] = dest_all
            return rc

        lax.fori_loop(0, na // LANES, p2_body, jnp.zeros((LANES,), jnp.int32))

        for v in range(nt_pad // LANES):
            tl = (v * LANES + lanes) * TM
            acc = jnp.full((LANES,), -1, jnp.int32)
            for e in range(E):
                off_e = jnp.sum(jnp.where(lanes == e, offset, 0))
                acc = acc + jnp.where(tl >= off_e, 1, 0)
            te_v[pl.ds(v * LANES, LANES)] = jnp.clip(acc, 0, E - 1)

        pltpu.sync_copy(rt_v, rt_hbm)
        pltpu.sync_copy(rw_v, rw_hbm)
        pltpu.sync_copy(eids_v, inv_hbm)
        pltpu.sync_copy(te_v, te_hbm)


def _dispatch(eids, wts, cap, nt_pad):
    na = eids.shape[0]
    kern = functools.partial(
        pl.kernel,
        out_type=(
            jax.ShapeDtypeStruct((cap,), jnp.int32),
            jax.ShapeDtypeStruct((cap,), jnp.float32),
            jax.ShapeDtypeStruct((na,), jnp.int32),
            jax.ShapeDtypeStruct((nt_pad,), jnp.int32),
        ),
        mesh=_mesh(),
        compiler_params=pltpu.CompilerParams(needs_layout_passes=False),
        scratch_types=[
            pltpu.VMEM((na,), jnp.int32),
            pltpu.VMEM((na,), jnp.float32),
            pltpu.VMEM((cap,), jnp.int32),
            pltpu.VMEM((cap,), jnp.float32),
            pltpu.VMEM((nt_pad,), jnp.int32),
        ],
    )
    return kern(functools.partial(_dispatch_body, na, cap, nt_pad))(eids, wts)


def _gather_body(rows_per_w, chunk, nchunk,
                 x_hbm, rt_hbm, xg_hbm, idx_v, rows_a, rows_b, sem_a, sem_b):
    base = _wid() * rows_per_w
    pltpu.sync_copy(rt_hbm.at[pl.ds(base, rows_per_w)], idx_v)
    bufs = [rows_a, rows_b]
    sems = [sem_a, sem_b]
    descs = [None, None]
    for c in range(nchunk):
        p = c % 2
        if descs[p] is not None:
            descs[p].wait()
        descs[p] = pltpu.async_copy(
            x_hbm.at[idx_v.at[pl.ds(c * chunk, chunk)]], bufs[p], sems[p])
        if c >= 1:
            q = (c - 1) % 2
            descs[q].wait()
            descs[q] = None
            pltpu.sync_copy(bufs[q],
                            xg_hbm.at[pl.ds(base + (c - 1) * chunk, chunk)])
    q = (nchunk - 1) % 2
    descs[q].wait()
    pltpu.sync_copy(bufs[q],
                    xg_hbm.at[pl.ds(base + (nchunk - 1) * chunk, chunk)])


def _gather(x, row_token, cap):
    t, dm = x.shape
    rows_per_w = cap // NW
    chunk = 40
    nchunk = rows_per_w // chunk
    kern = functools.partial(
        pl.kernel,
        out_type=jax.ShapeDtypeStruct((cap, dm), jnp.float32),
        mesh=_mesh(),
        compiler_params=pltpu.CompilerParams(needs_layout_passes=False),
        scratch_types=[
            pltpu.VMEM((rows_per_w,), jnp.int32),
            pltpu.VMEM((chunk, dm), jnp.float32),
            pltpu.VMEM((chunk, dm), jnp.float32),
            pltpu.SemaphoreType.DMA,
            pltpu.SemaphoreType.DMA,
        ],
    )
    return kern(
        functools.partial(_gather_body, rows_per_w, chunk, nchunk))(
            x, row_token)


def _ffn_body(nf, te_ref, xg_ref, w1_ref, w3_ref, w2_ref, rw_ref, out_ref,
              acc_ref):
    j = pl.program_id(0)
    i = pl.program_id(1)
    x = xg_ref[...]
    a = jnp.dot(x, w1_ref[0], preferred_element_type=jnp.float32)
    b = jnp.dot(x, w3_ref[0], preferred_element_type=jnp.float32)
    h = (a / (1.0 + jnp.exp(-a))) * b
    contrib = jnp.dot(h, w2_ref[0], preferred_element_type=jnp.float32)
    sl = pl.ds(i * TM, TM)

    @pl.when(j < nf - 1)
    def _():
        @pl.when(j == 0)
        def _():
            acc_ref[sl, :] = contrib.astype(jnp.bfloat16)

        @pl.when(j > 0)
        def _():
            acc_ref[sl, :] = (acc_ref[sl, :].astype(jnp.float32)
                              + contrib).astype(jnp.bfloat16)

    @pl.when(j == nf - 1)
    def _():
        out_ref[...] = ((acc_ref[sl, :].astype(jnp.float32) + contrib)
                        * rw_ref[0][0][:, None])


def _ffn(xg, w1, w2, w3, row_weight, tile_expert, nt):
    cap, dm = xg.shape
    dff = w1.shape[2]
    fb = 1792
    nf = dff // fb
    rw3 = row_weight.reshape(nt, 1, TM)
    grid_spec = pltpu.PrefetchScalarGridSpec(
        num_scalar_prefetch=1,
        grid=(nf, nt),
        in_specs=[
            pl.BlockSpec((TM, dm), lambda j, i, te: (i, 0)),
            pl.BlockSpec((1, dm, fb), lambda j, i, te: (te[i], 0, j)),
            pl.BlockSpec((1, dm, fb), lambda j, i, te: (te[i], 0, j)),
            pl.BlockSpec((1, fb, dm), lambda j, i, te: (te[i], j, 0)),
            pl.BlockSpec((1, 1, TM), lambda j, i, te: (i, 0, 0)),
        ],
        out_specs=pl.BlockSpec(
            (TM, dm), lambda j, i, te: (jnp.where(j == nf - 1, i, 0), 0)),
        scratch_shapes=[pltpu.VMEM((cap, dm), jnp.bfloat16)],
    )
    return pl.pallas_call(
        functools.partial(_ffn_body, nf),
        grid_spec=grid_spec,
        out_shape=jax.ShapeDtypeStruct((cap, dm), jnp.float32),
    )(tile_expert, xg, w1, w3, w2, rw3)


def _combine_body(t, tok_chunk, nchunk,
                  y_hbm, inv_hbm, out_hbm, inv_v, ybuf_a, ybuf_b, obuf,
                  sem_a, sem_b):
    wid = _wid()
    tok_per_w = t // NW
    dm = obuf.shape[1]
    pltpu.sync_copy(inv_hbm.at[pl.ds(wid * tok_per_w * K, tok_per_w * K)],
                    inv_v)
    bufs = [ybuf_a, ybuf_b]
    sems = [sem_a, sem_b]

    def issue(c):
        return pltpu.async_copy(
            y_hbm.at[inv_v.at[pl.ds(c * tok_chunk * K, tok_chunk * K)]],
            bufs[c % 2], sems[c % 2])

    descs = [issue(0), issue(1)]
    for c in range(nchunk):
        descs[c % 2].wait()
        ybuf = bufs[c % 2]

        def rloop(r, cr):
            def gloop(g, cg):
                s = pl.ds(pl.multiple_of(g * LANES, LANES), LANES)
                obuf[r, s] = ybuf[2 * r, s] + ybuf[2 * r + 1, s]
                return cg
            lax.fori_loop(0, dm // LANES, gloop, 0)
            return cr

        lax.fori_loop(0, tok_chunk, rloop, 0)
        if c + 2 < nchunk:
            descs[c % 2] = issue(c + 2)
        pltpu.sync_copy(
            obuf, out_hbm.at[pl.ds(wid * tok_per_w + c * tok_chunk, tok_chunk)])


def _combine(y, inv, t):
    dm = y.shape[1]
    tok_chunk = 16
    nchunk = t // NW // tok_chunk
    kern = functools.partial(
        pl.kernel,
        out_type=jax.ShapeDtypeStruct((t, dm), jnp.float32),
        mesh=_mesh(),
        compiler_params=pltpu.CompilerParams(needs_layout_passes=False),
        scratch_types=[
            pltpu.VMEM((t // NW * K,), jnp.int32),
            pltpu.VMEM((tok_chunk * K, dm), jnp.float32),
            pltpu.VMEM((tok_chunk * K, dm), jnp.float32),
            pltpu.VMEM((tok_chunk, dm), jnp.float32),
            pltpu.SemaphoreType.DMA,
            pltpu.SemaphoreType.DMA,
        ],
    )
    return kern(functools.partial(_combine_body, t, tok_chunk, nchunk))(y, inv)


def kernel(hidden_states, gate_w, w1, w2, w3):
    batch, seq, dm = hidden_states.shape
    t = batch * seq
    x = hidden_states.reshape(t, dm)
    nt = (t * K) // TM + E
    nt_pad = ((nt + LANES - 1) // LANES) * LANES
    cap = nt * TM

    logits_p, ids_p, wts_p = _router(x, gate_w)
    router_logits = logits_p[:, :E]
    eids = ids_p[:, :K].reshape(t * K)
    wv = wts_p[:, :K].reshape(t * K)

    row_token, row_weight, inv, te = _dispatch(eids, wv, cap, nt_pad)
    xg = _gather(x, row_token, cap)
    y = _ffn(xg, w1, w2, w3, row_weight, te[:nt], nt)
    out = _combine(y, inv, t)
    return (out.reshape(batch, seq, dm), router_logits)
